# baseline (speedup 1.0000x reference)
import functools

import jax
import jax.numpy as jnp
from jax import lax
from jax.experimental import pallas as pl
from jax.experimental.pallas import tpu as pltpu
from jax.experimental.pallas import tpu_sc as plsc

N = 10000
E = 320000
H = 128
D_EDGE = 16
L = 3

NB = 1000
BE = 2000
PREC = lax.Precision.HIGHEST
EPREC = lax.Precision.DEFAULT

NC = 2
NS = 16
NW = NC * NS
EH = E
EPW = EH // NW
C = 80
NCH = EPW // C
NBUF = 5
NBUF_S = 3
NPAD = 10240
RPT = NPAD // NS


def _silu(v):
    return v * (1.0 / (1.0 + jnp.exp(-v)))


def _embed_body(x_ref, wemb_ref, bemb_ref, w1s_ref, w1d_ref,
                h_ref, p_ref, q_ref):
    h = jnp.dot(x_ref[...], wemb_ref[...], precision=PREC) + bemb_ref[...]
    h_ref[...] = h
    p_ref[...] = jnp.dot(h, w1s_ref[...], precision=PREC)
    q_ref[...] = jnp.dot(h, w1d_ref[...], precision=PREC)


def _edge_body(g_ref, ea_ref, w1e_ref, b1_ref, w2_ref, b2_ref, m2_ref):
    t = g_ref[...] + jnp.dot(ea_ref[...], w1e_ref[...], precision=EPREC) \
        + b1_ref[...]
    m = _silu(t)
    t2 = jnp.dot(m, w2_ref[...], precision=EPREC) + b2_ref[...]
    m2_ref[...] = _silu(t2)


def _update_body(h_ref, a0_ref, a1_ref,
                 wu1h_ref, wu1a_ref, bu1_ref,
                 wu2_ref, bu2_ref, w1s_ref, w1d_ref,
                 h_out, p_out, q_out):
    h = h_ref[...]
    agg = a0_ref[...] + a1_ref[...]
    u = jnp.dot(h, wu1h_ref[...], precision=PREC) \
        + jnp.dot(agg, wu1a_ref[...], precision=PREC) + bu1_ref[...]
    u = _silu(u)
    hn = h + jnp.dot(u, wu2_ref[...], precision=PREC) + bu2_ref[...]
    h_out[...] = hn
    p_out[...] = jnp.dot(hn, w1s_ref[...], precision=PREC)
    q_out[...] = jnp.dot(hn, w1d_ref[...], precision=PREC)


def _final_body(h_ref, a0_ref, a1_ref,
                wu1h_ref, wu1a_ref, bu1_ref,
                wu2_ref, bu2_ref, wp1_ref, bp1_ref, wp2_ref, bp2_ref,
                out_ref):
    h = h_ref[...]
    agg = a0_ref[...] + a1_ref[...]
    u = jnp.dot(h, wu1h_ref[...], precision=PREC) \
        + jnp.dot(agg, wu1a_ref[...], precision=PREC) + bu1_ref[...]
    u = _silu(u)
    hn = h + jnp.dot(u, wu2_ref[...], precision=PREC) + bu2_ref[...]
    o = jnp.dot(hn, wp1_ref[...], precision=PREC) + bp1_ref[...]
    o = jnp.dot(o, wp2_ref[...], precision=PREC) + bp2_ref[...]
    mx = jnp.max(o, axis=1, keepdims=True)
    lse = jnp.log(jnp.sum(jnp.exp(o - mx), axis=1, keepdims=True)) + mx
    out_ref[...] = o - lse


def _full(shape):
    return pl.BlockSpec(shape, lambda i: (0,) * len(shape))


def _rows(shape):
    return pl.BlockSpec(shape, lambda i: (i,) + (0,) * (len(shape) - 1))


_node_sds = jax.ShapeDtypeStruct((N, H), jnp.float32)

_embed = pl.pallas_call(
    _embed_body,
    grid=(N // NB,),
    in_specs=[_rows((NB, H)), _full((H, H)), _full((1, H)),
              _full((H, H)), _full((H, H))],
    out_specs=[_rows((NB, H))] * 3,
    out_shape=[_node_sds] * 3,
)

_edge_mlp = pl.pallas_call(
    _edge_body,
    grid=(EH // BE,),
    in_specs=[_rows((BE, H)), _rows((BE, D_EDGE)), _full((D_EDGE, H)),
              _full((1, H)), _full((H, H)), _full((1, H))],
    out_specs=_rows((BE, H)),
    out_shape=jax.ShapeDtypeStruct((EH, H), jnp.float32),
)

_update = pl.pallas_call(
    _update_body,
    grid=(N // NB,),
    in_specs=[_rows((NB, H))] * 3 +
             [_full((H, H)), _full((H, H)), _full((1, H)),
              _full((H, H)), _full((1, H)), _full((H, H)), _full((H, H))],
    out_specs=[_rows((NB, H))] * 3,
    out_shape=[_node_sds] * 3,
)

_final = pl.pallas_call(
    _final_body,
    grid=(N // NB,),
    in_specs=[_rows((NB, H))] * 3 +
             [_full((H, H)), _full((H, H)), _full((1, H)),
              _full((H, H)), _full((1, H)),
              _full((H, H)), _full((1, H)), _full((H, H)), _full((1, H))],
    out_specs=_rows((NB, H)),
    out_shape=_node_sds,
)


def _sc_gather_body(p_hbm, q_hbm, src3_hbm, dst3_hbm, g_hbm,
                    sidx, didx, bufs, gsems, wsems):
    wid = lax.axis_index("c") * NS + lax.axis_index("s")
    base = wid * EPW
    pltpu.sync_copy(src3_hbm.at[wid], sidx)
    pltpu.sync_copy(dst3_hbm.at[wid], didx)

    def fire_p(k, b):
        @pl.when(k >= NBUF)
        def _():
            pltpu.make_async_copy(
                bufs[b], g_hbm.at[pl.ds(base, C)], wsems[b]).wait()
        pltpu.async_copy(p_hbm.at[sidx.at[k]], bufs[b], gsems[b])

    def fire_q(k, b):
        pltpu.make_async_copy(
            p_hbm.at[sidx.at[k]], bufs[b], gsems[b]).wait()
        pltpu.async_copy(q_hbm.at[didx.at[k]], bufs[b], gsems[b], add=True)

    def fire_w(k, b):
        pltpu.make_async_copy(
            q_hbm.at[didx.at[k]], bufs[b], gsems[b]).wait()
        pltpu.async_copy(bufs[b], g_hbm.at[pl.ds(base + k * C, C)], wsems[b])

    def round_(r, carry):
        for b in range(NBUF):
            fire_p(r * NBUF + b, b)
        for b in range(NBUF):
            fire_q(r * NBUF + b, b)
        for b in range(NBUF):
            fire_w(r * NBUF + b, b)
        return carry

    lax.fori_loop(0, NCH // NBUF, round_, 0)
    for b in range(NBUF):
        pltpu.make_async_copy(
            bufs[b], g_hbm.at[pl.ds(base, C)], wsems[b]).wait()


@functools.cache
def _get_sc_gather():
    return pl.kernel(
        _sc_gather_body,
        out_type=jax.ShapeDtypeStruct((EH, H), jnp.float32),
        mesh=plsc.VectorSubcoreMesh(core_axis_name="c", subcore_axis_name="s",
                                    num_cores=NC, num_subcores=NS),
        scratch_types=[
            pltpu.VMEM((NCH, C), jnp.int32),
            pltpu.VMEM((NCH, C), jnp.int32),
            [pltpu.VMEM((C, H), jnp.float32)] * NBUF,
            [pltpu.SemaphoreType.DMA] * NBUF,
            [pltpu.SemaphoreType.DMA] * NBUF,
        ],
    )


def _sc_scatter_body(m2_hbm, dst3_hbm, out0_hbm, out1_hbm,
                     didx, mbufs, msems, ssems, agg):
    cid = lax.axis_index("c")
    sid = lax.axis_index("s")
    wid = cid * NS + sid
    base = wid * EPW
    pltpu.sync_copy(dst3_hbm.at[wid], didx)

    def zrow(r, carry):
        for j in range(H // 16):
            mbufs[0][r, pl.ds(j * 16, 16)] = jnp.zeros((16,), jnp.float32)
        return carry

    lax.fori_loop(0, C, zrow, 0)
    for k in range(RPT // C):
        pltpu.sync_copy(mbufs[0], agg.at[pl.ds(sid * RPT + k * C, C)])
    plsc.subcore_barrier()

    def fire_load(k, b):
        @pl.when(k >= NBUF_S)
        def _():
            pltpu.make_async_copy(
                mbufs[b], agg.at[didx.at[k]], ssems[b]).wait()
        pltpu.async_copy(m2_hbm.at[pl.ds(base + k * C, C)], mbufs[b],
                         msems[b])

    def fire_scatter(k, b):
        pltpu.make_async_copy(
            m2_hbm.at[pl.ds(base, C)], mbufs[b], msems[b]).wait()
        pltpu.async_copy(mbufs[b], agg.at[didx.at[k]], ssems[b], add=True)

    def round_(r, carry):
        for b in range(NBUF_S):
            fire_load(r * NBUF_S + b, b)
        for b in range(NBUF_S):
            fire_scatter(r * NBUF_S + b, b)
        return carry

    nfull = NCH // NBUF_S
    lax.fori_loop(0, nfull, round_, 0)
    for b in range(NCH - nfull * NBUF_S):
        fire_load(nfull * NBUF_S + b, b)
        fire_scatter(nfull * NBUF_S + b, b)
    for b in range(NBUF_S):
        pltpu.make_async_copy(mbufs[b], agg.at[didx.at[0]], ssems[b]).wait()
    plsc.subcore_barrier()

    rows = pl.ds(sid * RPT, RPT)

    @pl.when(cid == 0)
    def _():
        pltpu.sync_copy(agg.at[rows], out0_hbm.at[rows])

    @pl.when(cid == 1)
    def _():
        pltpu.sync_copy(agg.at[rows], out1_hbm.at[rows])


@functools.cache
def _get_sc_scatter():
    return pl.kernel(
        _sc_scatter_body,
        out_type=[jax.ShapeDtypeStruct((NPAD, H), jnp.float32)] * 2,
        mesh=plsc.VectorSubcoreMesh(core_axis_name="c", subcore_axis_name="s",
                                    num_cores=NC, num_subcores=NS),
        scratch_types=[
            pltpu.VMEM((NCH, C), jnp.int32),
            [pltpu.VMEM((C, H), jnp.float32)] * NBUF_S,
            [pltpu.SemaphoreType.DMA] * NBUF_S,
            [pltpu.SemaphoreType.DMA] * NBUF_S,
            pltpu.VMEM_SHARED((NPAD, H), jnp.float32),
        ],
    )


def kernel(x, edge_index, batch, edge_attr, W_emb, b_emb, Wm1, bm1, Wm2, bm2,
           Wu1, bu1, Wu2, bu2, W_p1, b_p1, W_p2, b_p2):
    src3 = edge_index[0].reshape(NW, NCH, C)
    dst3 = edge_index[1].reshape(NW, NCH, C)
    r1 = lambda b: b.reshape(1, H)

    h, p, q = _embed(x, W_emb, r1(b_emb), Wm1[0, :H], Wm1[0, H:2 * H])
    out = None
    for l in range(L):
        g = _get_sc_gather()(p, q, src3, dst3)
        m2 = _edge_mlp(g, edge_attr, Wm1[l, 2 * H:], r1(bm1[l]),
                       Wm2[l], r1(bm2[l]))
        parts = _get_sc_scatter()(m2, dst3)
        if l < L - 1:
            h, p, q = _update(h, *parts,
                              Wu1[l, :H], Wu1[l, H:], r1(bu1[l]),
                              Wu2[l], r1(bu2[l]),
                              Wm1[l + 1, :H], Wm1[l + 1, H:2 * H])
        else:
            out = _final(h, *parts,
                         Wu1[l, :H], Wu1[l, H:], r1(bu1[l]),
                         Wu2[l], r1(bu2[l]),
                         W_p1, r1(b_p1), W_p2, r1(b_p2))
    return out


# speedup vs baseline: 1.0783x; 1.0783x over previous
import functools

import jax
import jax.numpy as jnp
from jax import lax
from jax.experimental import pallas as pl
from jax.experimental.pallas import tpu as pltpu
from jax.experimental.pallas import tpu_sc as plsc

N = 10000
E = 320000
H = 128
D_EDGE = 16
L = 3

NB = 1000
BE = 2000
PREC = lax.Precision.HIGHEST
EPREC = lax.Precision.DEFAULT

NC = 2
NS = 16
NW = NC * NS
EH = E // 2
EPW = EH // NW
C = 40
NCH = EPW // C
NBUF = 5
NBUF_S = 5
NPAD = 10240
RPT = NPAD // NS


def _silu(v):
    return v * (1.0 / (1.0 + jnp.exp(-v)))


def _embed_body(x_ref, wemb_ref, bemb_ref, w1s_ref, w1d_ref,
                h_ref, p_ref, q_ref):
    h = jnp.dot(x_ref[...], wemb_ref[...], precision=PREC) + bemb_ref[...]
    h_ref[...] = h
    p_ref[...] = jnp.dot(h, w1s_ref[...], precision=PREC)
    q_ref[...] = jnp.dot(h, w1d_ref[...], precision=PREC)


def _edge_body(g_ref, ea_ref, w1e_ref, b1_ref, w2_ref, b2_ref, m2_ref):
    t = g_ref[...] + jnp.dot(ea_ref[...], w1e_ref[...], precision=EPREC) \
        + b1_ref[...]
    m = _silu(t)
    t2 = jnp.dot(m, w2_ref[...], precision=EPREC) + b2_ref[...]
    m2_ref[...] = _silu(t2)


def _update_body(h_ref, a0_ref, a1_ref, a2_ref, a3_ref,
                 wu1h_ref, wu1a_ref, bu1_ref,
                 wu2_ref, bu2_ref, w1s_ref, w1d_ref,
                 h_out, p_out, q_out):
    h = h_ref[...]
    agg = (a0_ref[...] + a1_ref[...]) + (a2_ref[...] + a3_ref[...])
    u = jnp.dot(h, wu1h_ref[...], precision=PREC) \
        + jnp.dot(agg, wu1a_ref[...], precision=PREC) + bu1_ref[...]
    u = _silu(u)
    hn = h + jnp.dot(u, wu2_ref[...], precision=PREC) + bu2_ref[...]
    h_out[...] = hn
    p_out[...] = jnp.dot(hn, w1s_ref[...], precision=PREC)
    q_out[...] = jnp.dot(hn, w1d_ref[...], precision=PREC)


def _final_body(h_ref, a0_ref, a1_ref, a2_ref, a3_ref,
                wu1h_ref, wu1a_ref, bu1_ref,
                wu2_ref, bu2_ref, wp1_ref, bp1_ref, wp2_ref, bp2_ref,
                out_ref):
    h = h_ref[...]
    agg = (a0_ref[...] + a1_ref[...]) + (a2_ref[...] + a3_ref[...])
    u = jnp.dot(h, wu1h_ref[...], precision=PREC) \
        + jnp.dot(agg, wu1a_ref[...], precision=PREC) + bu1_ref[...]
    u = _silu(u)
    hn = h + jnp.dot(u, wu2_ref[...], precision=PREC) + bu2_ref[...]
    o = jnp.dot(hn, wp1_ref[...], precision=PREC) + bp1_ref[...]
    o = jnp.dot(o, wp2_ref[...], precision=PREC) + bp2_ref[...]
    mx = jnp.max(o, axis=1, keepdims=True)
    lse = jnp.log(jnp.sum(jnp.exp(o - mx), axis=1, keepdims=True)) + mx
    out_ref[...] = o - lse


def _full(shape):
    return pl.BlockSpec(shape, lambda i: (0,) * len(shape))


def _rows(shape):
    return pl.BlockSpec(shape, lambda i: (i,) + (0,) * (len(shape) - 1))


_node_sds = jax.ShapeDtypeStruct((N, H), jnp.float32)

_embed = pl.pallas_call(
    _embed_body,
    grid=(N // NB,),
    in_specs=[_rows((NB, H)), _full((H, H)), _full((1, H)),
              _full((H, H)), _full((H, H))],
    out_specs=[_rows((NB, H))] * 3,
    out_shape=[_node_sds] * 3,
)

_edge_mlp = pl.pallas_call(
    _edge_body,
    grid=(EH // BE,),
    in_specs=[_rows((BE, H)), _rows((BE, D_EDGE)), _full((D_EDGE, H)),
              _full((1, H)), _full((H, H)), _full((1, H))],
    out_specs=_rows((BE, H)),
    out_shape=jax.ShapeDtypeStruct((EH, H), jnp.float32),
)

_update = pl.pallas_call(
    _update_body,
    grid=(N // NB,),
    in_specs=[_rows((NB, H))] * 5 +
             [_full((H, H)), _full((H, H)), _full((1, H)),
              _full((H, H)), _full((1, H)), _full((H, H)), _full((H, H))],
    out_specs=[_rows((NB, H))] * 3,
    out_shape=[_node_sds] * 3,
)

_final = pl.pallas_call(
    _final_body,
    grid=(N // NB,),
    in_specs=[_rows((NB, H))] * 5 +
             [_full((H, H)), _full((H, H)), _full((1, H)),
              _full((H, H)), _full((1, H)),
              _full((H, H)), _full((1, H)), _full((H, H)), _full((1, H))],
    out_specs=_rows((NB, H)),
    out_shape=_node_sds,
)


def _sc_gather_body(p_hbm, q_hbm, src3_hbm, dst3_hbm, g_hbm,
                    sidx, didx, bufs, gsems, wsems):
    wid = lax.axis_index("c") * NS + lax.axis_index("s")
    base = wid * EPW
    pltpu.sync_copy(src3_hbm.at[wid], sidx)
    pltpu.sync_copy(dst3_hbm.at[wid], didx)

    def fire_p(k, b):
        @pl.when(k >= NBUF)
        def _():
            pltpu.make_async_copy(
                bufs[b], g_hbm.at[pl.ds(base, C)], wsems[b]).wait()
        pltpu.async_copy(p_hbm.at[sidx.at[k]], bufs[b], gsems[b])

    def fire_q(k, b):
        pltpu.make_async_copy(
            p_hbm.at[sidx.at[k]], bufs[b], gsems[b]).wait()
        pltpu.async_copy(q_hbm.at[didx.at[k]], bufs[b], gsems[b], add=True)

    def fire_w(k, b):
        pltpu.make_async_copy(
            q_hbm.at[didx.at[k]], bufs[b], gsems[b]).wait()
        pltpu.async_copy(bufs[b], g_hbm.at[pl.ds(base + k * C, C)], wsems[b])

    def round_(r, carry):
        for b in range(NBUF):
            fire_p(r * NBUF + b, b)
        for b in range(NBUF):
            fire_q(r * NBUF + b, b)
        for b in range(NBUF):
            fire_w(r * NBUF + b, b)
        return carry

    lax.fori_loop(0, NCH // NBUF, round_, 0)
    for b in range(NBUF):
        pltpu.make_async_copy(
            bufs[b], g_hbm.at[pl.ds(base, C)], wsems[b]).wait()


@functools.cache
def _get_sc_gather():
    return pl.kernel(
        _sc_gather_body,
        out_type=jax.ShapeDtypeStruct((EH, H), jnp.float32),
        mesh=plsc.VectorSubcoreMesh(core_axis_name="c", subcore_axis_name="s",
                                    num_cores=NC, num_subcores=NS),
        scratch_types=[
            pltpu.VMEM((NCH, C), jnp.int32),
            pltpu.VMEM((NCH, C), jnp.int32),
            [pltpu.VMEM((C, H), jnp.float32)] * NBUF,
            [pltpu.SemaphoreType.DMA] * NBUF,
            [pltpu.SemaphoreType.DMA] * NBUF,
        ],
    )


def _sc_scatter_body(m2_hbm, dst3_hbm, out0_hbm, out1_hbm,
                     didx, mbufs, msems, ssems, agg):
    cid = lax.axis_index("c")
    sid = lax.axis_index("s")
    wid = cid * NS + sid
    base = wid * EPW
    pltpu.sync_copy(dst3_hbm.at[wid], didx)

    def zrow(r, carry):
        for j in range(H // 16):
            mbufs[0][r, pl.ds(j * 16, 16)] = jnp.zeros((16,), jnp.float32)
        return carry

    lax.fori_loop(0, C, zrow, 0)
    for k in range(RPT // C):
        pltpu.async_copy(mbufs[0], agg.at[pl.ds(sid * RPT + k * C, C)],
                         msems[0])
    for k in range(RPT // C):
        pltpu.make_async_copy(
            mbufs[0], agg.at[pl.ds(sid * RPT, C)], msems[0]).wait()
    plsc.subcore_barrier()

    def fire_load(k, b):
        @pl.when(k >= NBUF_S)
        def _():
            pltpu.make_async_copy(
                mbufs[b], agg.at[didx.at[k]], ssems[b]).wait()
        pltpu.async_copy(m2_hbm.at[pl.ds(base + k * C, C)], mbufs[b],
                         msems[b])

    def fire_scatter(k, b):
        pltpu.make_async_copy(
            m2_hbm.at[pl.ds(base, C)], mbufs[b], msems[b]).wait()
        pltpu.async_copy(mbufs[b], agg.at[didx.at[k]], ssems[b], add=True)

    def round_(r, carry):
        for b in range(NBUF_S):
            fire_load(r * NBUF_S + b, b)
        for b in range(NBUF_S):
            fire_scatter(r * NBUF_S + b, b)
        return carry

    nfull = NCH // NBUF_S
    lax.fori_loop(0, nfull, round_, 0)
    for b in range(NCH - nfull * NBUF_S):
        fire_load(nfull * NBUF_S + b, b)
        fire_scatter(nfull * NBUF_S + b, b)
    for b in range(NBUF_S):
        pltpu.make_async_copy(mbufs[b], agg.at[didx.at[0]], ssems[b]).wait()
    plsc.subcore_barrier()

    rows = pl.ds(sid * RPT, RPT)

    @pl.when(cid == 0)
    def _():
        pltpu.sync_copy(agg.at[rows], out0_hbm.at[rows])

    @pl.when(cid == 1)
    def _():
        pltpu.sync_copy(agg.at[rows], out1_hbm.at[rows])


@functools.cache
def _get_sc_scatter():
    return pl.kernel(
        _sc_scatter_body,
        out_type=[jax.ShapeDtypeStruct((NPAD, H), jnp.float32)] * 2,
        mesh=plsc.VectorSubcoreMesh(core_axis_name="c", subcore_axis_name="s",
                                    num_cores=NC, num_subcores=NS),
        scratch_types=[
            pltpu.VMEM((NCH, C), jnp.int32),
            [pltpu.VMEM((C, H), jnp.float32)] * NBUF_S,
            [pltpu.SemaphoreType.DMA] * NBUF_S,
            [pltpu.SemaphoreType.DMA] * NBUF_S,
            pltpu.VMEM_SHARED((NPAD, H), jnp.float32),
        ],
    )


def kernel(x, edge_index, batch, edge_attr, W_emb, b_emb, Wm1, bm1, Wm2, bm2,
           Wu1, bu1, Wu2, bu2, W_p1, b_p1, W_p2, b_p2):
    src3 = edge_index[0].reshape(2, NW, NCH, C)
    dst3 = edge_index[1].reshape(2, NW, NCH, C)
    ea2 = edge_attr.reshape(2, EH, D_EDGE)
    r1 = lambda b: b.reshape(1, H)

    h, p, q = _embed(x, W_emb, r1(b_emb), Wm1[0, :H], Wm1[0, H:2 * H])
    out = None
    for l in range(L):
        parts = []
        for hf in range(2):
            g = _get_sc_gather()(p, q, src3[hf], dst3[hf])
            m2 = _edge_mlp(g, ea2[hf], Wm1[l, 2 * H:], r1(bm1[l]),
                           Wm2[l], r1(bm2[l]))
            parts += _get_sc_scatter()(m2, dst3[hf])
        if l < L - 1:
            h, p, q = _update(h, *parts,
                              Wu1[l, :H], Wu1[l, H:], r1(bu1[l]),
                              Wu2[l], r1(bu2[l]),
                              Wm1[l + 1, :H], Wm1[l + 1, H:2 * H])
        else:
            out = _final(h, *parts,
                         Wu1[l, :H], Wu1[l, H:], r1(bu1[l]),
                         Wu2[l], r1(bu2[l]),
                         W_p1, r1(b_p1), W_p2, r1(b_p2))
    return out


# speedup vs baseline: 1.0876x; 1.0087x over previous
import functools

import jax
import jax.numpy as jnp
from jax import lax
from jax.experimental import pallas as pl
from jax.experimental.pallas import tpu as pltpu
from jax.experimental.pallas import tpu_sc as plsc

N = 10000
E = 320000
H = 128
D_EDGE = 16
L = 3

NB = 1000
BE = 2000
PREC = lax.Precision.HIGHEST
EPREC = lax.Precision.DEFAULT

NC = 2
NS = 16
NW = NC * NS
EH = E // 2
EPW = EH // NW
C = 40
NCH = EPW // C
NBUF = 10
NBUF_S = 5
NPAD = 10240
RPT = NPAD // NS


def _silu(v):
    return v * (1.0 / (1.0 + jnp.exp(-v)))


def _embed_body(x_ref, wemb_ref, bemb_ref, w1s_ref, w1d_ref,
                h_ref, p_ref, q_ref):
    h = jnp.dot(x_ref[...], wemb_ref[...], precision=PREC) + bemb_ref[...]
    h_ref[...] = h
    p_ref[...] = jnp.dot(h, w1s_ref[...], precision=PREC)
    q_ref[...] = jnp.dot(h, w1d_ref[...], precision=PREC)


def _edge_body(g_ref, ea_ref, w1e_ref, b1_ref, w2_ref, b2_ref, m2_ref):
    t = g_ref[...] + jnp.dot(ea_ref[...], w1e_ref[...], precision=EPREC) \
        + b1_ref[...]
    m = _silu(t)
    t2 = jnp.dot(m, w2_ref[...], precision=EPREC) + b2_ref[...]
    m2_ref[...] = _silu(t2)


def _update_body(h_ref, a0_ref, a1_ref, a2_ref, a3_ref,
                 wu1h_ref, wu1a_ref, bu1_ref,
                 wu2_ref, bu2_ref, w1s_ref, w1d_ref,
                 h_out, p_out, q_out):
    h = h_ref[...]
    agg = (a0_ref[...] + a1_ref[...]) + (a2_ref[...] + a3_ref[...])
    u = jnp.dot(h, wu1h_ref[...], precision=PREC) \
        + jnp.dot(agg, wu1a_ref[...], precision=PREC) + bu1_ref[...]
    u = _silu(u)
    hn = h + jnp.dot(u, wu2_ref[...], precision=PREC) + bu2_ref[...]
    h_out[...] = hn
    p_out[...] = jnp.dot(hn, w1s_ref[...], precision=PREC)
    q_out[...] = jnp.dot(hn, w1d_ref[...], precision=PREC)


def _final_body(h_ref, a0_ref, a1_ref, a2_ref, a3_ref,
                wu1h_ref, wu1a_ref, bu1_ref,
                wu2_ref, bu2_ref, wp1_ref, bp1_ref, wp2_ref, bp2_ref,
                out_ref):
    h = h_ref[...]
    agg = (a0_ref[...] + a1_ref[...]) + (a2_ref[...] + a3_ref[...])
    u = jnp.dot(h, wu1h_ref[...], precision=PREC) \
        + jnp.dot(agg, wu1a_ref[...], precision=PREC) + bu1_ref[...]
    u = _silu(u)
    hn = h + jnp.dot(u, wu2_ref[...], precision=PREC) + bu2_ref[...]
    o = jnp.dot(hn, wp1_ref[...], precision=PREC) + bp1_ref[...]
    o = jnp.dot(o, wp2_ref[...], precision=PREC) + bp2_ref[...]
    mx = jnp.max(o, axis=1, keepdims=True)
    lse = jnp.log(jnp.sum(jnp.exp(o - mx), axis=1, keepdims=True)) + mx
    out_ref[...] = o - lse


def _full(shape):
    return pl.BlockSpec(shape, lambda i: (0,) * len(shape))


def _rows(shape):
    return pl.BlockSpec(shape, lambda i: (i,) + (0,) * (len(shape) - 1))


_node_sds = jax.ShapeDtypeStruct((N, H), jnp.float32)

_embed = pl.pallas_call(
    _embed_body,
    grid=(N // NB,),
    in_specs=[_rows((NB, H)), _full((H, H)), _full((1, H)),
              _full((H, H)), _full((H, H))],
    out_specs=[_rows((NB, H))] * 3,
    out_shape=[_node_sds] * 3,
)

_edge_mlp = pl.pallas_call(
    _edge_body,
    grid=(EH // BE,),
    in_specs=[_rows((BE, H)), _rows((BE, D_EDGE)), _full((D_EDGE, H)),
              _full((1, H)), _full((H, H)), _full((1, H))],
    out_specs=_rows((BE, H)),
    out_shape=jax.ShapeDtypeStruct((EH, H), jnp.float32),
)

_update = pl.pallas_call(
    _update_body,
    grid=(N // NB,),
    in_specs=[_rows((NB, H))] * 5 +
             [_full((H, H)), _full((H, H)), _full((1, H)),
              _full((H, H)), _full((1, H)), _full((H, H)), _full((H, H))],
    out_specs=[_rows((NB, H))] * 3,
    out_shape=[_node_sds] * 3,
)

_final = pl.pallas_call(
    _final_body,
    grid=(N // NB,),
    in_specs=[_rows((NB, H))] * 5 +
             [_full((H, H)), _full((H, H)), _full((1, H)),
              _full((H, H)), _full((1, H)),
              _full((H, H)), _full((1, H)), _full((H, H)), _full((1, H))],
    out_specs=_rows((NB, H)),
    out_shape=_node_sds,
)


def _sc_gather_body(p_hbm, q_hbm, src3_hbm, dst3_hbm, g_hbm,
                    sidx, didx, bufs, gsems, wsems):
    wid = lax.axis_index("c") * NS + lax.axis_index("s")
    base = wid * EPW
    pltpu.sync_copy(src3_hbm.at[wid], sidx)
    pltpu.sync_copy(dst3_hbm.at[wid], didx)

    def fire_p(k, b):
        @pl.when(k >= NBUF)
        def _():
            pltpu.make_async_copy(
                bufs[b], g_hbm.at[pl.ds(base, C)], wsems[b]).wait()
        pltpu.async_copy(p_hbm.at[sidx.at[k]], bufs[b], gsems[b])

    def fire_q(k, b):
        pltpu.make_async_copy(
            p_hbm.at[sidx.at[k]], bufs[b], gsems[b]).wait()
        pltpu.async_copy(q_hbm.at[didx.at[k]], bufs[b], gsems[b], add=True)

    def fire_w(k, b):
        pltpu.make_async_copy(
            q_hbm.at[didx.at[k]], bufs[b], gsems[b]).wait()
        pltpu.async_copy(bufs[b], g_hbm.at[pl.ds(base + k * C, C)], wsems[b])

    def round_(r, carry):
        for b in range(NBUF):
            fire_p(r * NBUF + b, b)
        for b in range(NBUF):
            fire_q(r * NBUF + b, b)
        for b in range(NBUF):
            fire_w(r * NBUF + b, b)
        return carry

    nfull = NCH // NBUF
    lax.fori_loop(0, nfull, round_, 0)
    rem = NCH - nfull * NBUF
    for b in range(rem):
        fire_p(nfull * NBUF + b, b)
    for b in range(rem):
        fire_q(nfull * NBUF + b, b)
    for b in range(rem):
        fire_w(nfull * NBUF + b, b)
    for b in range(NBUF):
        pltpu.make_async_copy(
            bufs[b], g_hbm.at[pl.ds(base, C)], wsems[b]).wait()


@functools.cache
def _get_sc_gather():
    return pl.kernel(
        _sc_gather_body,
        out_type=jax.ShapeDtypeStruct((EH, H), jnp.float32),
        mesh=plsc.VectorSubcoreMesh(core_axis_name="c", subcore_axis_name="s",
                                    num_cores=NC, num_subcores=NS),
        scratch_types=[
            pltpu.VMEM((NCH, C), jnp.int32),
            pltpu.VMEM((NCH, C), jnp.int32),
            [pltpu.VMEM((C, H), jnp.float32)] * NBUF,
            [pltpu.SemaphoreType.DMA] * NBUF,
            [pltpu.SemaphoreType.DMA] * NBUF,
        ],
    )


def _sc_scatter_body(m2_hbm, dst3_hbm, out0_hbm, out1_hbm,
                     didx, mbufs, msems, ssems, agg):
    cid = lax.axis_index("c")
    sid = lax.axis_index("s")
    wid = cid * NS + sid
    base = wid * EPW
    pltpu.sync_copy(dst3_hbm.at[wid], didx)

    def zrow(r, carry):
        for j in range(H // 16):
            mbufs[0][r, pl.ds(j * 16, 16)] = jnp.zeros((16,), jnp.float32)
        return carry

    lax.fori_loop(0, C, zrow, 0)
    for k in range(RPT // C):
        pltpu.async_copy(mbufs[0], agg.at[pl.ds(sid * RPT + k * C, C)],
                         msems[0])
    for k in range(RPT // C):
        pltpu.make_async_copy(
            mbufs[0], agg.at[pl.ds(sid * RPT, C)], msems[0]).wait()
    plsc.subcore_barrier()

    def fire_load(k, b):
        @pl.when(k >= NBUF_S)
        def _():
            pltpu.make_async_copy(
                mbufs[b], agg.at[didx.at[k]], ssems[b]).wait()
        pltpu.async_copy(m2_hbm.at[pl.ds(base + k * C, C)], mbufs[b],
                         msems[b])

    def fire_scatter(k, b):
        pltpu.make_async_copy(
            m2_hbm.at[pl.ds(base, C)], mbufs[b], msems[b]).wait()
        pltpu.async_copy(mbufs[b], agg.at[didx.at[k]], ssems[b], add=True)

    def round_(r, carry):
        for b in range(NBUF_S):
            fire_load(r * NBUF_S + b, b)
        for b in range(NBUF_S):
            fire_scatter(r * NBUF_S + b, b)
        return carry

    nfull = NCH // NBUF_S
    lax.fori_loop(0, nfull, round_, 0)
    for b in range(NCH - nfull * NBUF_S):
        fire_load(nfull * NBUF_S + b, b)
        fire_scatter(nfull * NBUF_S + b, b)
    for b in range(NBUF_S):
        pltpu.make_async_copy(mbufs[b], agg.at[didx.at[0]], ssems[b]).wait()
    plsc.subcore_barrier()

    rows = pl.ds(sid * RPT, RPT)

    @pl.when(cid == 0)
    def _():
        pltpu.sync_copy(agg.at[rows], out0_hbm.at[rows])

    @pl.when(cid == 1)
    def _():
        pltpu.sync_copy(agg.at[rows], out1_hbm.at[rows])


@functools.cache
def _get_sc_scatter():
    return pl.kernel(
        _sc_scatter_body,
        out_type=[jax.ShapeDtypeStruct((NPAD, H), jnp.float32)] * 2,
        mesh=plsc.VectorSubcoreMesh(core_axis_name="c", subcore_axis_name="s",
                                    num_cores=NC, num_subcores=NS),
        scratch_types=[
            pltpu.VMEM((NCH, C), jnp.int32),
            [pltpu.VMEM((C, H), jnp.float32)] * NBUF_S,
            [pltpu.SemaphoreType.DMA] * NBUF_S,
            [pltpu.SemaphoreType.DMA] * NBUF_S,
            pltpu.VMEM_SHARED((NPAD, H), jnp.float32),
        ],
    )


def kernel(x, edge_index, batch, edge_attr, W_emb, b_emb, Wm1, bm1, Wm2, bm2,
           Wu1, bu1, Wu2, bu2, W_p1, b_p1, W_p2, b_p2):
    src3 = edge_index[0].reshape(2, NW, NCH, C)
    dst3 = edge_index[1].reshape(2, NW, NCH, C)
    ea2 = edge_attr.reshape(2, EH, D_EDGE)
    r1 = lambda b: b.reshape(1, H)

    h, p, q = _embed(x, W_emb, r1(b_emb), Wm1[0, :H], Wm1[0, H:2 * H])
    out = None
    for l in range(L):
        parts = []
        for hf in range(2):
            g = _get_sc_gather()(p, q, src3[hf], dst3[hf])
            m2 = _edge_mlp(g, ea2[hf], Wm1[l, 2 * H:], r1(bm1[l]),
                           Wm2[l], r1(bm2[l]))
            parts += _get_sc_scatter()(m2, dst3[hf])
        if l < L - 1:
            h, p, q = _update(h, *parts,
                              Wu1[l, :H], Wu1[l, H:], r1(bu1[l]),
                              Wu2[l], r1(bu2[l]),
                              Wm1[l + 1, :H], Wm1[l + 1, H:2 * H])
        else:
            out = _final(h, *parts,
                         Wu1[l, :H], Wu1[l, H:], r1(bu1[l]),
                         Wu2[l], r1(bu2[l]),
                         W_p1, r1(b_p1), W_p2, r1(b_p2))
    return out


# speedup vs baseline: 1.1338x; 1.0425x over previous
import functools

import jax
import jax.numpy as jnp
from jax import lax
from jax.experimental import pallas as pl
from jax.experimental.pallas import tpu as pltpu
from jax.experimental.pallas import tpu_sc as plsc

N = 10000
E = 320000
H = 128
D_EDGE = 16
L = 3

NB = 1000
BE = 4000
PREC = lax.Precision.HIGHEST
EPREC = lax.Precision.DEFAULT

NC = 2
NS = 16
NW = NC * NS
EH = E // 2
EPW = EH // NW
C = 40
NCH = EPW // C
NBUF = 10
NBUF_S = 6
NPAD = 10240
RPT = NPAD // NS


def _silu(v):
    return v * (1.0 / (1.0 + jnp.exp(-v)))


def _embed_body(x_ref, wemb_ref, bemb_ref, w1s_ref, w1d_ref,
                h_ref, p_ref, q_ref):
    h = jnp.dot(x_ref[...], wemb_ref[...], precision=PREC) + bemb_ref[...]
    h_ref[...] = h
    p_ref[...] = jnp.dot(h, w1s_ref[...], precision=PREC)
    q_ref[...] = jnp.dot(h, w1d_ref[...], precision=PREC)


def _edge_body(g_ref, ea_ref, w1e_ref, b1_ref, w2_ref, b2_ref, m2_ref):
    t = g_ref[...] + jnp.dot(ea_ref[...], w1e_ref[...], precision=EPREC) \
        + b1_ref[...]
    m = _silu(t)
    t2 = jnp.dot(m, w2_ref[...], precision=EPREC) + b2_ref[...]
    m2_ref[...] = _silu(t2)


def _update_body(h_ref, a0_ref, a1_ref, a2_ref, a3_ref,
                 wu1h_ref, wu1a_ref, bu1_ref,
                 wu2_ref, bu2_ref, w1s_ref, w1d_ref,
                 h_out, p_out, q_out):
    h = h_ref[...]
    agg = (a0_ref[...] + a1_ref[...]) + (a2_ref[...] + a3_ref[...])
    u = jnp.dot(h, wu1h_ref[...], precision=PREC) \
        + jnp.dot(agg, wu1a_ref[...], precision=PREC) + bu1_ref[...]
    u = _silu(u)
    hn = h + jnp.dot(u, wu2_ref[...], precision=PREC) + bu2_ref[...]
    h_out[...] = hn
    p_out[...] = jnp.dot(hn, w1s_ref[...], precision=PREC)
    q_out[...] = jnp.dot(hn, w1d_ref[...], precision=PREC)


def _final_body(h_ref, a0_ref, a1_ref, a2_ref, a3_ref,
                wu1h_ref, wu1a_ref, bu1_ref,
                wu2_ref, bu2_ref, wp1_ref, bp1_ref, wp2_ref, bp2_ref,
                out_ref):
    h = h_ref[...]
    agg = (a0_ref[...] + a1_ref[...]) + (a2_ref[...] + a3_ref[...])
    u = jnp.dot(h, wu1h_ref[...], precision=PREC) \
        + jnp.dot(agg, wu1a_ref[...], precision=PREC) + bu1_ref[...]
    u = _silu(u)
    hn = h + jnp.dot(u, wu2_ref[...], precision=PREC) + bu2_ref[...]
    o = jnp.dot(hn, wp1_ref[...], precision=PREC) + bp1_ref[...]
    o = jnp.dot(o, wp2_ref[...], precision=PREC) + bp2_ref[...]
    mx = jnp.max(o, axis=1, keepdims=True)
    lse = jnp.log(jnp.sum(jnp.exp(o - mx), axis=1, keepdims=True)) + mx
    out_ref[...] = o - lse


def _full(shape):
    return pl.BlockSpec(shape, lambda i: (0,) * len(shape))


def _rows(shape):
    return pl.BlockSpec(shape, lambda i: (i,) + (0,) * (len(shape) - 1))


_node_sds = jax.ShapeDtypeStruct((N, H), jnp.float32)

_embed = pl.pallas_call(
    _embed_body,
    grid=(N // NB,),
    in_specs=[_rows((NB, H)), _full((H, H)), _full((1, H)),
              _full((H, H)), _full((H, H))],
    out_specs=[_rows((NB, H))] * 3,
    out_shape=[_node_sds] * 3,
)

_edge_mlp = pl.pallas_call(
    _edge_body,
    grid=(EH // BE,),
    in_specs=[_rows((BE, H)), _rows((BE, D_EDGE)), _full((D_EDGE, H)),
              _full((1, H)), _full((H, H)), _full((1, H))],
    out_specs=_rows((BE, H)),
    out_shape=jax.ShapeDtypeStruct((EH, H), jnp.float32),
)

_update = pl.pallas_call(
    _update_body,
    grid=(N // NB,),
    in_specs=[_rows((NB, H))] * 5 +
             [_full((H, H)), _full((H, H)), _full((1, H)),
              _full((H, H)), _full((1, H)), _full((H, H)), _full((H, H))],
    out_specs=[_rows((NB, H))] * 3,
    out_shape=[_node_sds] * 3,
)

_final = pl.pallas_call(
    _final_body,
    grid=(N // NB,),
    in_specs=[_rows((NB, H))] * 5 +
             [_full((H, H)), _full((H, H)), _full((1, H)),
              _full((H, H)), _full((1, H)),
              _full((H, H)), _full((1, H)), _full((H, H)), _full((1, H))],
    out_specs=_rows((NB, H)),
    out_shape=_node_sds,
)


def _sc_gather_body(p_hbm, q_hbm, src3_hbm, dst3_hbm, g_hbm,
                    sidx, didx, bufs, gsems, wsems):
    wid = lax.axis_index("c") * NS + lax.axis_index("s")
    base = wid * EPW
    pltpu.sync_copy(src3_hbm.at[wid], sidx)
    pltpu.sync_copy(dst3_hbm.at[wid], didx)

    def fire_p(k, b):
        @pl.when(k >= NBUF)
        def _():
            pltpu.make_async_copy(
                bufs[b], g_hbm.at[pl.ds(base, C)], wsems[b]).wait()
        pltpu.async_copy(p_hbm.at[sidx.at[k]], bufs[b], gsems[b])

    def fire_q(k, b):
        pltpu.make_async_copy(
            p_hbm.at[sidx.at[k]], bufs[b], gsems[b]).wait()
        pltpu.async_copy(q_hbm.at[didx.at[k]], bufs[b], gsems[b], add=True)

    def fire_w(k, b):
        pltpu.make_async_copy(
            q_hbm.at[didx.at[k]], bufs[b], gsems[b]).wait()
        pltpu.async_copy(bufs[b], g_hbm.at[pl.ds(base + k * C, C)], wsems[b])

    def round_(r, carry):
        for b in range(NBUF):
            fire_p(r * NBUF + b, b)
        for b in range(NBUF):
            fire_q(r * NBUF + b, b)
        for b in range(NBUF):
            fire_w(r * NBUF + b, b)
        return carry

    nfull = NCH // NBUF
    lax.fori_loop(0, nfull, round_, 0)
    rem = NCH - nfull * NBUF
    for b in range(rem):
        fire_p(nfull * NBUF + b, b)
    for b in range(rem):
        fire_q(nfull * NBUF + b, b)
    for b in range(rem):
        fire_w(nfull * NBUF + b, b)
    for b in range(NBUF):
        pltpu.make_async_copy(
            bufs[b], g_hbm.at[pl.ds(base, C)], wsems[b]).wait()


@functools.cache
def _get_sc_gather():
    return pl.kernel(
        _sc_gather_body,
        out_type=jax.ShapeDtypeStruct((EH, H), jnp.float32),
        mesh=plsc.VectorSubcoreMesh(core_axis_name="c", subcore_axis_name="s",
                                    num_cores=NC, num_subcores=NS),
        scratch_types=[
            pltpu.VMEM((NCH, C), jnp.int32),
            pltpu.VMEM((NCH, C), jnp.int32),
            [pltpu.VMEM((C, H), jnp.float32)] * NBUF,
            [pltpu.SemaphoreType.DMA] * NBUF,
            [pltpu.SemaphoreType.DMA] * NBUF,
        ],
    )


def _sc_scatter_body(m2_hbm, dst3_hbm, out0_hbm, out1_hbm,
                     didx, mbufs, msems, ssems, agg):
    cid = lax.axis_index("c")
    sid = lax.axis_index("s")
    wid = cid * NS + sid
    base = wid * EPW
    pltpu.sync_copy(dst3_hbm.at[wid], didx)

    def zrow(r, carry):
        for j in range(H // 16):
            mbufs[0][r, pl.ds(j * 16, 16)] = jnp.zeros((16,), jnp.float32)
        return carry

    lax.fori_loop(0, C, zrow, 0)
    for k in range(RPT // C):
        pltpu.async_copy(mbufs[0], agg.at[pl.ds(sid * RPT + k * C, C)],
                         msems[0])
    for k in range(RPT // C):
        pltpu.make_async_copy(
            mbufs[0], agg.at[pl.ds(sid * RPT, C)], msems[0]).wait()
    plsc.subcore_barrier()

    def fire_load(k, b):
        @pl.when(k >= NBUF_S)
        def _():
            pltpu.make_async_copy(
                mbufs[b], agg.at[didx.at[k]], ssems[b]).wait()
        pltpu.async_copy(m2_hbm.at[pl.ds(base + k * C, C)], mbufs[b],
                         msems[b])

    def fire_scatter(k, b):
        pltpu.make_async_copy(
            m2_hbm.at[pl.ds(base, C)], mbufs[b], msems[b]).wait()
        pltpu.async_copy(mbufs[b], agg.at[didx.at[k]], ssems[b], add=True)

    def round_(r, carry):
        for b in range(NBUF_S):
            fire_load(r * NBUF_S + b, b)
        for b in range(NBUF_S):
            fire_scatter(r * NBUF_S + b, b)
        return carry

    nfull = NCH // NBUF_S
    lax.fori_loop(0, nfull, round_, 0)
    for b in range(NCH - nfull * NBUF_S):
        fire_load(nfull * NBUF_S + b, b)
        fire_scatter(nfull * NBUF_S + b, b)
    for b in range(NBUF_S):
        pltpu.make_async_copy(mbufs[b], agg.at[didx.at[0]], ssems[b]).wait()
    plsc.subcore_barrier()

    rows = pl.ds(sid * RPT, RPT)

    @pl.when(cid == 0)
    def _():
        pltpu.sync_copy(agg.at[rows], out0_hbm.at[rows])

    @pl.when(cid == 1)
    def _():
        pltpu.sync_copy(agg.at[rows], out1_hbm.at[rows])


@functools.cache
def _get_sc_scatter():
    return pl.kernel(
        _sc_scatter_body,
        out_type=[jax.ShapeDtypeStruct((NPAD, H), jnp.float32)] * 2,
        mesh=plsc.VectorSubcoreMesh(core_axis_name="c", subcore_axis_name="s",
                                    num_cores=NC, num_subcores=NS),
        scratch_types=[
            pltpu.VMEM((NCH, C), jnp.int32),
            [pltpu.VMEM((C, H), jnp.float32)] * NBUF_S,
            [pltpu.SemaphoreType.DMA] * NBUF_S,
            [pltpu.SemaphoreType.DMA] * NBUF_S,
            pltpu.VMEM_SHARED((NPAD, H), jnp.float32),
        ],
    )


def kernel(x, edge_index, batch, edge_attr, W_emb, b_emb, Wm1, bm1, Wm2, bm2,
           Wu1, bu1, Wu2, bu2, W_p1, b_p1, W_p2, b_p2):
    src3 = edge_index[0].reshape(2, NW, NCH, C)
    dst3 = edge_index[1].reshape(2, NW, NCH, C)
    ea2 = edge_attr.reshape(2, EH, D_EDGE)
    r1 = lambda b: b.reshape(1, H)

    h, p, q = _embed(x, W_emb, r1(b_emb), Wm1[0, :H], Wm1[0, H:2 * H])
    out = None
    for l in range(L):
        parts = []
        for hf in range(2):
            g = _get_sc_gather()(p, q, src3[hf], dst3[hf])
            m2 = _edge_mlp(g, ea2[hf], Wm1[l, 2 * H:], r1(bm1[l]),
                           Wm2[l], r1(bm2[l]))
            parts += _get_sc_scatter()(m2, dst3[hf])
        if l < L - 1:
            h, p, q = _update(h, *parts,
                              Wu1[l, :H], Wu1[l, H:], r1(bu1[l]),
                              Wu2[l], r1(bu2[l]),
                              Wm1[l + 1, :H], Wm1[l + 1, H:2 * H])
        else:
            out = _final(h, *parts,
                         Wu1[l, :H], Wu1[l, H:], r1(bu1[l]),
                         Wu2[l], r1(bu2[l]),
                         W_p1, r1(b_p1), W_p2, r1(b_p2))
    return out


# speedup vs baseline: 1.1841x; 1.0444x over previous
import functools

import jax
import jax.numpy as jnp
from jax import lax
from jax.experimental import pallas as pl
from jax.experimental.pallas import tpu as pltpu
from jax.experimental.pallas import tpu_sc as plsc

N = 10000
E = 320000
H = 128
D_EDGE = 16
L = 3

NB = 2000
BE = 8000
PREC = lax.Precision.HIGHEST
EPREC = lax.Precision.DEFAULT

NC = 2
NS = 16
NW = NC * NS
EH = E // 2
EPW = EH // NW
C = 40
NCH = EPW // C
NBUF = 10
NBUF_S = 6
NPAD = 10240
RPT = NPAD // NS


def _silu(v):
    return v * (1.0 / (1.0 + jnp.exp(-v)))


def _embed_body(x_ref, wemb_ref, bemb_ref, w1s_ref, w1d_ref,
                h_ref, p_ref, q_ref):
    h = jnp.dot(x_ref[...], wemb_ref[...], precision=PREC) + bemb_ref[...]
    h_ref[...] = h
    p_ref[...] = jnp.dot(h, w1s_ref[...], precision=PREC)
    q_ref[...] = jnp.dot(h, w1d_ref[...], precision=PREC)


def _edge_body(g_ref, ea_ref, w1e_ref, b1_ref, w2_ref, b2_ref, m2_ref):
    t = g_ref[...] + jnp.dot(ea_ref[...], w1e_ref[...], precision=EPREC) \
        + b1_ref[...]
    m = _silu(t)
    t2 = jnp.dot(m, w2_ref[...], precision=EPREC) + b2_ref[...]
    m2_ref[...] = _silu(t2)


def _update_body(h_ref, a0_ref, a1_ref, a2_ref, a3_ref,
                 wu1h_ref, wu1a_ref, bu1_ref,
                 wu2_ref, bu2_ref, w1s_ref, w1d_ref,
                 h_out, p_out, q_out):
    h = h_ref[...]
    agg = (a0_ref[...] + a1_ref[...]) + (a2_ref[...] + a3_ref[...])
    u = jnp.dot(h, wu1h_ref[...], precision=PREC) \
        + jnp.dot(agg, wu1a_ref[...], precision=PREC) + bu1_ref[...]
    u = _silu(u)
    hn = h + jnp.dot(u, wu2_ref[...], precision=PREC) + bu2_ref[...]
    h_out[...] = hn
    p_out[...] = jnp.dot(hn, w1s_ref[...], precision=PREC)
    q_out[...] = jnp.dot(hn, w1d_ref[...], precision=PREC)


def _final_body(h_ref, a0_ref, a1_ref, a2_ref, a3_ref,
                wu1h_ref, wu1a_ref, bu1_ref,
                wu2_ref, bu2_ref, wp1_ref, bp1_ref, wp2_ref, bp2_ref,
                out_ref):
    h = h_ref[...]
    agg = (a0_ref[...] + a1_ref[...]) + (a2_ref[...] + a3_ref[...])
    u = jnp.dot(h, wu1h_ref[...], precision=PREC) \
        + jnp.dot(agg, wu1a_ref[...], precision=PREC) + bu1_ref[...]
    u = _silu(u)
    hn = h + jnp.dot(u, wu2_ref[...], precision=PREC) + bu2_ref[...]
    o = jnp.dot(hn, wp1_ref[...], precision=PREC) + bp1_ref[...]
    o = jnp.dot(o, wp2_ref[...], precision=PREC) + bp2_ref[...]
    mx = jnp.max(o, axis=1, keepdims=True)
    lse = jnp.log(jnp.sum(jnp.exp(o - mx), axis=1, keepdims=True)) + mx
    out_ref[...] = o - lse


def _full(shape):
    return pl.BlockSpec(shape, lambda i: (0,) * len(shape))


def _rows(shape):
    return pl.BlockSpec(shape, lambda i: (i,) + (0,) * (len(shape) - 1))


_node_sds = jax.ShapeDtypeStruct((N, H), jnp.float32)

_embed = pl.pallas_call(
    _embed_body,
    grid=(N // NB,),
    in_specs=[_rows((NB, H)), _full((H, H)), _full((1, H)),
              _full((H, H)), _full((H, H))],
    out_specs=[_rows((NB, H))] * 3,
    out_shape=[_node_sds] * 3,
)

_edge_mlp = pl.pallas_call(
    _edge_body,
    grid=(EH // BE,),
    in_specs=[_rows((BE, H)), _rows((BE, D_EDGE)), _full((D_EDGE, H)),
              _full((1, H)), _full((H, H)), _full((1, H))],
    out_specs=_rows((BE, H)),
    out_shape=jax.ShapeDtypeStruct((EH, H), jnp.float32),
)

_update = pl.pallas_call(
    _update_body,
    grid=(N // NB,),
    in_specs=[_rows((NB, H))] * 5 +
             [_full((H, H)), _full((H, H)), _full((1, H)),
              _full((H, H)), _full((1, H)), _full((H, H)), _full((H, H))],
    out_specs=[_rows((NB, H))] * 3,
    out_shape=[_node_sds] * 3,
)

_final = pl.pallas_call(
    _final_body,
    grid=(N // NB,),
    in_specs=[_rows((NB, H))] * 5 +
             [_full((H, H)), _full((H, H)), _full((1, H)),
              _full((H, H)), _full((1, H)),
              _full((H, H)), _full((1, H)), _full((H, H)), _full((1, H))],
    out_specs=_rows((NB, H)),
    out_shape=_node_sds,
)


def _sc_gather_body(p_hbm, q_hbm, src3_hbm, dst3_hbm, g_hbm,
                    sidx, didx, bufs, gsems, wsems):
    wid = lax.axis_index("c") * NS + lax.axis_index("s")
    base = wid * EPW
    pltpu.sync_copy(src3_hbm.at[wid], sidx)
    pltpu.sync_copy(dst3_hbm.at[wid], didx)

    def fire_p(k, b):
        @pl.when(k >= NBUF)
        def _():
            pltpu.make_async_copy(
                bufs[b], g_hbm.at[pl.ds(base, C)], wsems[b]).wait()
        pltpu.async_copy(p_hbm.at[sidx.at[k]], bufs[b], gsems[b])

    def fire_q(k, b):
        pltpu.make_async_copy(
            p_hbm.at[sidx.at[k]], bufs[b], gsems[b]).wait()
        pltpu.async_copy(q_hbm.at[didx.at[k]], bufs[b], gsems[b], add=True)

    def fire_w(k, b):
        pltpu.make_async_copy(
            q_hbm.at[didx.at[k]], bufs[b], gsems[b]).wait()
        pltpu.async_copy(bufs[b], g_hbm.at[pl.ds(base + k * C, C)], wsems[b])

    def round_(r, carry):
        for b in range(NBUF):
            fire_p(r * NBUF + b, b)
        for b in range(NBUF):
            fire_q(r * NBUF + b, b)
        for b in range(NBUF):
            fire_w(r * NBUF + b, b)
        return carry

    nfull = NCH // NBUF
    lax.fori_loop(0, nfull, round_, 0)
    rem = NCH - nfull * NBUF
    for b in range(rem):
        fire_p(nfull * NBUF + b, b)
    for b in range(rem):
        fire_q(nfull * NBUF + b, b)
    for b in range(rem):
        fire_w(nfull * NBUF + b, b)
    for b in range(NBUF):
        pltpu.make_async_copy(
            bufs[b], g_hbm.at[pl.ds(base, C)], wsems[b]).wait()


@functools.cache
def _get_sc_gather():
    return pl.kernel(
        _sc_gather_body,
        out_type=jax.ShapeDtypeStruct((EH, H), jnp.float32),
        mesh=plsc.VectorSubcoreMesh(core_axis_name="c", subcore_axis_name="s",
                                    num_cores=NC, num_subcores=NS),
        scratch_types=[
            pltpu.VMEM((NCH, C), jnp.int32),
            pltpu.VMEM((NCH, C), jnp.int32),
            [pltpu.VMEM((C, H), jnp.float32)] * NBUF,
            [pltpu.SemaphoreType.DMA] * NBUF,
            [pltpu.SemaphoreType.DMA] * NBUF,
        ],
    )


def _sc_scatter_body(m2_hbm, dst3_hbm, out0_hbm, out1_hbm,
                     didx, mbufs, msems, ssems, agg):
    cid = lax.axis_index("c")
    sid = lax.axis_index("s")
    wid = cid * NS + sid
    base = wid * EPW
    pltpu.sync_copy(dst3_hbm.at[wid], didx)

    def zrow(r, carry):
        for j in range(H // 16):
            mbufs[0][r, pl.ds(j * 16, 16)] = jnp.zeros((16,), jnp.float32)
        return carry

    lax.fori_loop(0, C, zrow, 0)
    for k in range(RPT // C):
        pltpu.async_copy(mbufs[0], agg.at[pl.ds(sid * RPT + k * C, C)],
                         msems[0])
    for k in range(RPT // C):
        pltpu.make_async_copy(
            mbufs[0], agg.at[pl.ds(sid * RPT, C)], msems[0]).wait()
    plsc.subcore_barrier()

    def fire_load(k, b):
        @pl.when(k >= NBUF_S)
        def _():
            pltpu.make_async_copy(
                mbufs[b], agg.at[didx.at[k]], ssems[b]).wait()
        pltpu.async_copy(m2_hbm.at[pl.ds(base + k * C, C)], mbufs[b],
                         msems[b])

    def fire_scatter(k, b):
        pltpu.make_async_copy(
            m2_hbm.at[pl.ds(base, C)], mbufs[b], msems[b]).wait()
        pltpu.async_copy(mbufs[b], agg.at[didx.at[k]], ssems[b], add=True)

    def round_(r, carry):
        for b in range(NBUF_S):
            fire_load(r * NBUF_S + b, b)
        for b in range(NBUF_S):
            fire_scatter(r * NBUF_S + b, b)
        return carry

    nfull = NCH // NBUF_S
    lax.fori_loop(0, nfull, round_, 0)
    for b in range(NCH - nfull * NBUF_S):
        fire_load(nfull * NBUF_S + b, b)
        fire_scatter(nfull * NBUF_S + b, b)
    for b in range(NBUF_S):
        pltpu.make_async_copy(mbufs[b], agg.at[didx.at[0]], ssems[b]).wait()
    plsc.subcore_barrier()

    rows = pl.ds(sid * RPT, RPT)

    @pl.when(cid == 0)
    def _():
        pltpu.sync_copy(agg.at[rows], out0_hbm.at[rows])

    @pl.when(cid == 1)
    def _():
        pltpu.sync_copy(agg.at[rows], out1_hbm.at[rows])


@functools.cache
def _get_sc_scatter():
    return pl.kernel(
        _sc_scatter_body,
        out_type=[jax.ShapeDtypeStruct((NPAD, H), jnp.float32)] * 2,
        mesh=plsc.VectorSubcoreMesh(core_axis_name="c", subcore_axis_name="s",
                                    num_cores=NC, num_subcores=NS),
        scratch_types=[
            pltpu.VMEM((NCH, C), jnp.int32),
            [pltpu.VMEM((C, H), jnp.float32)] * NBUF_S,
            [pltpu.SemaphoreType.DMA] * NBUF_S,
            [pltpu.SemaphoreType.DMA] * NBUF_S,
            pltpu.VMEM_SHARED((NPAD, H), jnp.float32),
        ],
    )


def kernel(x, edge_index, batch, edge_attr, W_emb, b_emb, Wm1, bm1, Wm2, bm2,
           Wu1, bu1, Wu2, bu2, W_p1, b_p1, W_p2, b_p2):
    src3 = edge_index[0].reshape(2, NW, NCH, C)
    dst3 = edge_index[1].reshape(2, NW, NCH, C)
    ea2 = edge_attr.reshape(2, EH, D_EDGE)
    r1 = lambda b: b.reshape(1, H)

    h, p, q = _embed(x, W_emb, r1(b_emb), Wm1[0, :H], Wm1[0, H:2 * H])
    out = None
    for l in range(L):
        parts = []
        for hf in range(2):
            g = _get_sc_gather()(p, q, src3[hf], dst3[hf])
            m2 = _edge_mlp(g, ea2[hf], Wm1[l, 2 * H:], r1(bm1[l]),
                           Wm2[l], r1(bm2[l]))
            parts += _get_sc_scatter()(m2, dst3[hf])
        if l < L - 1:
            h, p, q = _update(h, *parts,
                              Wu1[l, :H], Wu1[l, H:], r1(bu1[l]),
                              Wu2[l], r1(bu2[l]),
                              Wm1[l + 1, :H], Wm1[l + 1, H:2 * H])
        else:
            out = _final(h, *parts,
                         Wu1[l, :H], Wu1[l, H:], r1(bu1[l]),
                         Wu2[l], r1(bu2[l]),
                         W_p1, r1(b_p1), W_p2, r1(b_p2))
    return out
